# Initial kernel scaffold; baseline (speedup 1.0000x reference)
import jax
import jax.numpy as jnp
from jax.experimental import pallas as pl


def kernel(x, edge_index, edge_attr, batch, W1_0, b1_0, W2_0, b2_0, W1_1, b1_1, W2_1, b2_1, W1_2, b1_2, W2_2, b2_2, Wc1, bc1, gamma, beta, Wc2, bc2):
    raise NotImplementedError("write your pallas kernel here")


# speedup vs baseline: 6.3893x; 6.3893x over previous
import functools

import jax
import jax.numpy as jnp
from jax import lax
from jax.experimental import pallas as pl
from jax.experimental.pallas import tpu as pltpu
from jax.experimental.pallas import tpu_sc as plsc

N = 10000
E = 320000
D = 128
G = 128

NC = 2
NS = 16
NW = NC * NS
EPW = E // NW
CH = 80
NCHUNK = EPW // CH
NPAD = 10240
RPT = NPAD // NS


def _sc_agg_kernel(x_hbm, src_hbm, dst_hbm, zeros_hbm, out_hbm,
                   src_v, dst_v, rows_v, acc_sh, sem):
    c = lax.axis_index("c")
    s = lax.axis_index("s")
    w = s * NC + c

    pltpu.sync_copy(zeros_hbm, acc_sh.at[pl.ds(s * RPT, RPT)])

    pltpu.sync_copy(src_hbm.at[w], src_v)
    pltpu.sync_copy(dst_hbm.at[w], dst_v)

    plsc.subcore_barrier()

    def body(j, carry):
        pltpu.async_copy(x_hbm.at[src_v.at[j]], rows_v, sem).wait()
        pltpu.sync_copy(rows_v, acc_sh.at[dst_v.at[j]], add=True)
        return carry

    lax.fori_loop(0, NCHUNK, body, 0)

    plsc.subcore_barrier()

    pltpu.sync_copy(acc_sh.at[pl.ds(s * RPT, RPT)],
                    out_hbm.at[pl.ds(c * NPAD + s * RPT, RPT)])


@functools.partial(jax.jit, static_argnames=())
def _sc_agg(x, src2d, dst2d, zeros):
    mesh = plsc.VectorSubcoreMesh(core_axis_name="c", subcore_axis_name="s",
                                  num_cores=NC, num_subcores=NS)
    f = pl.kernel(
        _sc_agg_kernel,
        out_type=jax.ShapeDtypeStruct((NC * NPAD, D), jnp.float32),
        mesh=mesh,
        scratch_types=[
            pltpu.VMEM((NCHUNK, CH), jnp.int32),
            pltpu.VMEM((NCHUNK, CH), jnp.int32),
            pltpu.VMEM((CH, D), jnp.float32),
            pltpu.VMEM_SHARED((NPAD, D), jnp.float32),
            pltpu.SemaphoreType.DMA,
        ],
    )
    return f(x, src2d, dst2d, zeros)


RB = 1000


def _mlp_kernel(h_ref, a0_ref, a1_ref, w1_ref, b1_ref, w2_ref, b2_ref, out_ref):
    z = h_ref[...] + a0_ref[...] + a1_ref[...]
    y = jnp.dot(z, w1_ref[...], preferred_element_type=jnp.float32) + b1_ref[...]
    y = jnp.maximum(y, 0.0)
    o = jnp.dot(y, w2_ref[...], preferred_element_type=jnp.float32) + b2_ref[...]
    out_ref[...] = jnp.maximum(o, 0.0)


def _mlp(h, a0, a1, w1, b1, w2, b2):
    grid = N // RB
    return pl.pallas_call(
        _mlp_kernel,
        grid=(grid,),
        in_specs=[
            pl.BlockSpec((RB, D), lambda i: (i, 0)),
            pl.BlockSpec((RB, D), lambda i: (i, 0)),
            pl.BlockSpec((RB, D), lambda i: (i, 0)),
            pl.BlockSpec((D, D), lambda i: (0, 0)),
            pl.BlockSpec((1, D), lambda i: (0, 0)),
            pl.BlockSpec((D, D), lambda i: (0, 0)),
            pl.BlockSpec((1, D), lambda i: (0, 0)),
        ],
        out_specs=pl.BlockSpec((RB, D), lambda i: (i, 0)),
        out_shape=jax.ShapeDtypeStruct((N, D), jnp.float32),
    )(h, a0, a1, w1, b1.reshape(1, D), w2, b2.reshape(1, D))


def _pool_kernel(batch_ref, h1_ref, h2_ref, h3_ref,
                 w1a_ref, w1b_ref, w1c_ref, bc1_ref,
                 gb_ref, wc2_ref, bc2_ref, out_ref,
                 acc1, acc2, acc3):
    i = pl.program_id(0)

    @pl.when(i == 0)
    def _init():
        acc1[...] = jnp.zeros_like(acc1)
        acc2[...] = jnp.zeros_like(acc2)
        acc3[...] = jnp.zeros_like(acc3)

    ids = batch_ref[0, 0, :]
    gids = lax.broadcasted_iota(jnp.int32, (G, RB), 0)
    onehot_t = (gids == ids[None, :]).astype(jnp.float32)
    acc1[...] += jnp.dot(onehot_t, h1_ref[...], preferred_element_type=jnp.float32)
    acc2[...] += jnp.dot(onehot_t, h2_ref[...], preferred_element_type=jnp.float32)
    acc3[...] += jnp.dot(onehot_t, h3_ref[...], preferred_element_type=jnp.float32)

    @pl.when(i == (N // RB) - 1)
    def _final():
        z = (jnp.dot(acc1[...], w1a_ref[...], preferred_element_type=jnp.float32)
             + jnp.dot(acc2[...], w1b_ref[...], preferred_element_type=jnp.float32)
             + jnp.dot(acc3[...], w1c_ref[...], preferred_element_type=jnp.float32)
             + bc1_ref[...])
        gamma = gb_ref[0:1, :]
        beta = gb_ref[1:2, :]
        z = gamma * z * (1.0 / jnp.sqrt(1.0 + 1e-5)) + beta
        z = jnp.maximum(z, 0.0)
        out_ref[...] = jnp.dot(z, wc2_ref[...], preferred_element_type=jnp.float32) + bc2_ref[...]


def _pool_classify(batch3d, h1, h2, h3, w1a, w1b, w1c, bc1, gb, wc2p, bc2p):
    grid = N // RB
    return pl.pallas_call(
        _pool_kernel,
        grid=(grid,),
        in_specs=[
            pl.BlockSpec((1, 1, RB), lambda i: (i, 0, 0)),
            pl.BlockSpec((RB, D), lambda i: (i, 0)),
            pl.BlockSpec((RB, D), lambda i: (i, 0)),
            pl.BlockSpec((RB, D), lambda i: (i, 0)),
            pl.BlockSpec((D, D), lambda i: (0, 0)),
            pl.BlockSpec((D, D), lambda i: (0, 0)),
            pl.BlockSpec((D, D), lambda i: (0, 0)),
            pl.BlockSpec((1, D), lambda i: (0, 0)),
            pl.BlockSpec((2, D), lambda i: (0, 0)),
            pl.BlockSpec((D, D), lambda i: (0, 0)),
            pl.BlockSpec((1, D), lambda i: (0, 0)),
        ],
        out_specs=pl.BlockSpec((G, D), lambda i: (0, 0)),
        out_shape=jax.ShapeDtypeStruct((G, D), jnp.float32),
        scratch_shapes=[
            pltpu.VMEM((G, D), jnp.float32),
            pltpu.VMEM((G, D), jnp.float32),
            pltpu.VMEM((G, D), jnp.float32),
        ],
    )(batch3d, h1, h2, h3, w1a, w1b, w1c, bc1, gb, wc2p, bc2p)


def kernel(x, edge_index, edge_attr, batch,
           W1_0, b1_0, W2_0, b2_0,
           W1_1, b1_1, W2_1, b2_1,
           W1_2, b1_2, W2_2, b2_2,
           Wc1, bc1, gamma, beta, Wc2, bc2):
    del edge_attr

    src3d = edge_index[0].reshape(NW, NCHUNK, CH)
    dst3d = edge_index[1].reshape(NW, NCHUNK, CH)
    zeros = jnp.zeros((RPT, D), dtype=jnp.float32)

    params = [(W1_0, b1_0, W2_0, b2_0),
              (W1_1, b1_1, W2_1, b2_1),
              (W1_2, b1_2, W2_2, b2_2)]

    h = x
    hs = []
    for (w1, b1, w2, b2) in params:
        agg = _sc_agg(h, src3d, dst3d, zeros)
        h = _mlp(h, agg[:N], agg[NPAD:NPAD + N], w1, b1, w2, b2)
        hs.append(h)

    batch3d = batch.reshape(N // RB, 1, RB)
    w1a = Wc1[0:D]
    w1b = Wc1[D:2 * D]
    w1c = Wc1[2 * D:]
    gb = jnp.stack([gamma, beta], axis=0)
    wc2p = jnp.zeros((D, D), jnp.float32).at[:, :Wc2.shape[1]].set(Wc2)
    bc2p = jnp.zeros((1, D), jnp.float32).at[0, :Wc2.shape[1]].set(bc2)

    out = _pool_classify(batch3d, hs[0], hs[1], hs[2],
                         w1a, w1b, w1c, bc1.reshape(1, D), gb, wc2p, bc2p)
    return out[:, :Wc2.shape[1]]


# speedup vs baseline: 9.3171x; 1.4582x over previous
import functools

import jax
import jax.numpy as jnp
from jax import lax
from jax.experimental import pallas as pl
from jax.experimental.pallas import tpu as pltpu
from jax.experimental.pallas import tpu_sc as plsc

N = 10000
E = 320000
D = 128
G = 128

NC = 2
NS = 16
NW = NC * NS
EPW = E // NW
CH = 125
NCHUNK = EPW // CH
NPAD = 10240
RPT = NPAD // NS


def _sc_agg_kernel(x_hbm, idx_hbm, zeros_hbm, out_hbm,
                   idx_v, rows_v, acc_sh, sem0, sem1, semi0, semi1):
    c = lax.axis_index("c")
    s = lax.axis_index("s")
    w = s * NC + c

    pltpu.sync_copy(zeros_hbm, acc_sh.at[pl.ds(s * RPT, RPT)])

    plsc.subcore_barrier()

    rows = (rows_v.at[pl.ds(0, CH)], rows_v.at[pl.ds(CH, CH)])
    islot = (idx_v.at[0], idx_v.at[1])
    gsem = (sem0, sem1)
    isem = (semi0, semi1)

    def stage_idx(j, p):
        pltpu.async_copy(idx_hbm.at[w].at[j], islot[p], isem[p])

    def wait_idx(j, p):
        pltpu.make_async_copy(idx_hbm.at[w].at[j], islot[p], isem[p]).wait()

    def gather(p):
        pltpu.async_copy(x_hbm.at[islot[p].at[0]], rows[p], gsem[p])

    def wait_gather(p):
        pltpu.make_async_copy(x_hbm.at[islot[p].at[0]], rows[p], gsem[p]).wait()

    def scatter(p):
        pltpu.sync_copy(rows[p], acc_sh.at[islot[p].at[1]], add=True)

    stage_idx(0, 0)
    wait_idx(0, 0)
    stage_idx(1, 1)
    gather(0)

    def step(j, p):
        q = 1 - p
        wait_gather(p)
        wait_idx(j + 1, q)
        gather(q)
        scatter(p)
        stage_idx(j + 2, p)

    def body(j2, carry):
        j = j2 * 2
        step(j, 0)
        step(j + 1, 1)
        return carry

    lax.fori_loop(0, NCHUNK // 2 - 1, body, 0)
    wait_gather(0)
    wait_idx(NCHUNK - 1, 1)
    gather(1)
    scatter(0)
    wait_gather(1)
    scatter(1)

    plsc.subcore_barrier()

    pltpu.sync_copy(acc_sh.at[pl.ds(s * RPT, RPT)],
                    out_hbm.at[pl.ds(c * NPAD + s * RPT, RPT)])


@functools.partial(jax.jit, static_argnames=())
def _sc_agg(x, idx4d, zeros):
    mesh = plsc.VectorSubcoreMesh(core_axis_name="c", subcore_axis_name="s",
                                  num_cores=NC, num_subcores=NS)
    f = pl.kernel(
        _sc_agg_kernel,
        out_type=jax.ShapeDtypeStruct((NC * NPAD, D), jnp.float32),
        mesh=mesh,
        scratch_types=[
            pltpu.VMEM((2, 2, CH), jnp.int32),
            pltpu.VMEM((2 * CH, D), jnp.float32),
            pltpu.VMEM_SHARED((NPAD, D), jnp.float32),
            pltpu.SemaphoreType.DMA,
            pltpu.SemaphoreType.DMA,
            pltpu.SemaphoreType.DMA,
            pltpu.SemaphoreType.DMA,
        ],
    )
    return f(x, idx4d, zeros)


RB = 1000


def _mlp_kernel(h_ref, a0_ref, a1_ref, w1_ref, b1_ref, w2_ref, b2_ref, out_ref):
    z = h_ref[...] + a0_ref[...] + a1_ref[...]
    y = jnp.dot(z, w1_ref[...], preferred_element_type=jnp.float32) + b1_ref[...]
    y = jnp.maximum(y, 0.0)
    o = jnp.dot(y, w2_ref[...], preferred_element_type=jnp.float32) + b2_ref[...]
    out_ref[...] = jnp.maximum(o, 0.0)


def _mlp(h, a0, a1, w1, b1, w2, b2):
    grid = N // RB
    return pl.pallas_call(
        _mlp_kernel,
        grid=(grid,),
        in_specs=[
            pl.BlockSpec((RB, D), lambda i: (i, 0)),
            pl.BlockSpec((RB, D), lambda i: (i, 0)),
            pl.BlockSpec((RB, D), lambda i: (i, 0)),
            pl.BlockSpec((D, D), lambda i: (0, 0)),
            pl.BlockSpec((1, D), lambda i: (0, 0)),
            pl.BlockSpec((D, D), lambda i: (0, 0)),
            pl.BlockSpec((1, D), lambda i: (0, 0)),
        ],
        out_specs=pl.BlockSpec((RB, D), lambda i: (i, 0)),
        out_shape=jax.ShapeDtypeStruct((N, D), jnp.float32),
    )(h, a0, a1, w1, b1.reshape(1, D), w2, b2.reshape(1, D))


def _pool_kernel(batch_ref, h1_ref, h2_ref, h3_ref,
                 w1a_ref, w1b_ref, w1c_ref, bc1_ref,
                 gb_ref, wc2_ref, bc2_ref, out_ref,
                 acc1, acc2, acc3):
    i = pl.program_id(0)

    @pl.when(i == 0)
    def _init():
        acc1[...] = jnp.zeros_like(acc1)
        acc2[...] = jnp.zeros_like(acc2)
        acc3[...] = jnp.zeros_like(acc3)

    ids = batch_ref[0, 0, :]
    gids = lax.broadcasted_iota(jnp.int32, (G, RB), 0)
    onehot_t = (gids == ids[None, :]).astype(jnp.float32)
    acc1[...] += jnp.dot(onehot_t, h1_ref[...], preferred_element_type=jnp.float32)
    acc2[...] += jnp.dot(onehot_t, h2_ref[...], preferred_element_type=jnp.float32)
    acc3[...] += jnp.dot(onehot_t, h3_ref[...], preferred_element_type=jnp.float32)

    @pl.when(i == (N // RB) - 1)
    def _final():
        z = (jnp.dot(acc1[...], w1a_ref[...], preferred_element_type=jnp.float32)
             + jnp.dot(acc2[...], w1b_ref[...], preferred_element_type=jnp.float32)
             + jnp.dot(acc3[...], w1c_ref[...], preferred_element_type=jnp.float32)
             + bc1_ref[...])
        gamma = gb_ref[0:1, :]
        beta = gb_ref[1:2, :]
        z = gamma * z * (1.0 / jnp.sqrt(1.0 + 1e-5)) + beta
        z = jnp.maximum(z, 0.0)
        out_ref[...] = jnp.dot(z, wc2_ref[...], preferred_element_type=jnp.float32) + bc2_ref[...]


def _pool_classify(batch3d, h1, h2, h3, w1a, w1b, w1c, bc1, gb, wc2p, bc2p):
    grid = N // RB
    return pl.pallas_call(
        _pool_kernel,
        grid=(grid,),
        in_specs=[
            pl.BlockSpec((1, 1, RB), lambda i: (i, 0, 0)),
            pl.BlockSpec((RB, D), lambda i: (i, 0)),
            pl.BlockSpec((RB, D), lambda i: (i, 0)),
            pl.BlockSpec((RB, D), lambda i: (i, 0)),
            pl.BlockSpec((D, D), lambda i: (0, 0)),
            pl.BlockSpec((D, D), lambda i: (0, 0)),
            pl.BlockSpec((D, D), lambda i: (0, 0)),
            pl.BlockSpec((1, D), lambda i: (0, 0)),
            pl.BlockSpec((2, D), lambda i: (0, 0)),
            pl.BlockSpec((D, D), lambda i: (0, 0)),
            pl.BlockSpec((1, D), lambda i: (0, 0)),
        ],
        out_specs=pl.BlockSpec((G, D), lambda i: (0, 0)),
        out_shape=jax.ShapeDtypeStruct((G, D), jnp.float32),
        scratch_shapes=[
            pltpu.VMEM((G, D), jnp.float32),
            pltpu.VMEM((G, D), jnp.float32),
            pltpu.VMEM((G, D), jnp.float32),
        ],
    )(batch3d, h1, h2, h3, w1a, w1b, w1c, bc1, gb, wc2p, bc2p)


def kernel(x, edge_index, edge_attr, batch,
           W1_0, b1_0, W2_0, b2_0,
           W1_1, b1_1, W2_1, b2_1,
           W1_2, b1_2, W2_2, b2_2,
           Wc1, bc1, gamma, beta, Wc2, bc2):
    del edge_attr

    src3d = edge_index[0].reshape(NW, NCHUNK, CH)
    dst3d = edge_index[1].reshape(NW, NCHUNK, CH)
    idx4d = jnp.stack([src3d, dst3d], axis=2)
    zeros = jnp.zeros((RPT, D), dtype=jnp.float32)

    params = [(W1_0, b1_0, W2_0, b2_0),
              (W1_1, b1_1, W2_1, b2_1),
              (W1_2, b1_2, W2_2, b2_2)]

    h = x
    hs = []
    for (w1, b1, w2, b2) in params:
        agg = _sc_agg(h, idx4d, zeros)
        h = _mlp(h, agg[:N], agg[NPAD:NPAD + N], w1, b1, w2, b2)
        hs.append(h)

    batch3d = batch.reshape(N // RB, 1, RB)
    w1a = Wc1[0:D]
    w1b = Wc1[D:2 * D]
    w1c = Wc1[2 * D:]
    gb = jnp.stack([gamma, beta], axis=0)
    wc2p = jnp.zeros((D, D), jnp.float32).at[:, :Wc2.shape[1]].set(Wc2)
    bc2p = jnp.zeros((1, D), jnp.float32).at[0, :Wc2.shape[1]].set(bc2)

    out = _pool_classify(batch3d, hs[0], hs[1], hs[2],
                         w1a, w1b, w1c, bc1.reshape(1, D), gb, wc2p, bc2p)
    return out[:, :Wc2.shape[1]]


# speedup vs baseline: 9.3801x; 1.0068x over previous
import functools

import jax
import jax.numpy as jnp
from jax import lax
from jax.experimental import pallas as pl
from jax.experimental.pallas import tpu as pltpu
from jax.experimental.pallas import tpu_sc as plsc

N = 10000
E = 320000
D = 128
G = 128

NC = 2
NS = 16
NW = NC * NS
EPW = E // NW
CH = 100
NCHUNK = EPW // CH
DEPTH = 3
NPAD = 10240
RPT = NPAD // NS


def _sc_agg_kernel(x_hbm, idx_hbm, zeros_hbm, out_hbm,
                   idx_v, rows_v, acc_sh,
                   gsem0, gsem1, gsem2, isem0, isem1, isem2):
    c = lax.axis_index("c")
    s = lax.axis_index("s")
    w = s * NC + c

    pltpu.sync_copy(zeros_hbm, acc_sh.at[pl.ds(s * RPT, RPT)])

    plsc.subcore_barrier()

    rows = tuple(rows_v.at[pl.ds(k * CH, CH)] for k in range(DEPTH))
    islot = tuple(idx_v.at[k] for k in range(DEPTH))
    gsem = (gsem0, gsem1, gsem2)
    isem = (isem0, isem1, isem2)

    def stage_idx(j, p):
        pltpu.async_copy(idx_hbm.at[w].at[j], islot[p], isem[p])

    def wait_idx(j, p):
        pltpu.make_async_copy(idx_hbm.at[w].at[j], islot[p], isem[p]).wait()

    def gather(p):
        pltpu.async_copy(x_hbm.at[islot[p].at[0]], rows[p], gsem[p])

    def wait_gather(p):
        pltpu.make_async_copy(x_hbm.at[islot[p].at[0]], rows[p], gsem[p]).wait()

    def scatter(p):
        pltpu.sync_copy(rows[p], acc_sh.at[islot[p].at[1]], add=True)

    stage_idx(0, 0)
    stage_idx(1, 1)
    wait_idx(0, 0)
    gather(0)
    wait_idx(1, 1)
    gather(1)
    stage_idx(2, 2)

    def step(j, a, do_gather=True, do_stage=True):
        if do_gather:
            cslot = (a + 2) % DEPTH
            wait_idx(j + 2, cslot)
            gather(cslot)
        wait_gather(a)
        scatter(a)
        if do_stage:
            stage_idx(j + 3, a)

    def body(j3, carry):
        j = j3 * DEPTH
        step(j, 0)
        step(j + 1, 1)
        step(j + 2, 2)
        return carry

    lax.fori_loop(0, (NCHUNK - 4) // DEPTH, body, 0)
    step(NCHUNK - 4, (NCHUNK - 4) % DEPTH)
    step(NCHUNK - 3, (NCHUNK - 3) % DEPTH, do_stage=False)
    step(NCHUNK - 2, (NCHUNK - 2) % DEPTH, do_gather=False, do_stage=False)
    step(NCHUNK - 1, (NCHUNK - 1) % DEPTH, do_gather=False, do_stage=False)

    plsc.subcore_barrier()

    pltpu.sync_copy(acc_sh.at[pl.ds(s * RPT, RPT)],
                    out_hbm.at[pl.ds(c * NPAD + s * RPT, RPT)])


@functools.partial(jax.jit, static_argnames=())
def _sc_agg(x, idx4d, zeros):
    mesh = plsc.VectorSubcoreMesh(core_axis_name="c", subcore_axis_name="s",
                                  num_cores=NC, num_subcores=NS)
    f = pl.kernel(
        _sc_agg_kernel,
        out_type=jax.ShapeDtypeStruct((NC * NPAD, D), jnp.float32),
        mesh=mesh,
        scratch_types=[
            pltpu.VMEM((DEPTH, 2, CH), jnp.int32),
            pltpu.VMEM((DEPTH * CH, D), jnp.float32),
            pltpu.VMEM_SHARED((NPAD, D), jnp.float32),
            pltpu.SemaphoreType.DMA,
            pltpu.SemaphoreType.DMA,
            pltpu.SemaphoreType.DMA,
            pltpu.SemaphoreType.DMA,
            pltpu.SemaphoreType.DMA,
            pltpu.SemaphoreType.DMA,
        ],
    )
    return f(x, idx4d, zeros)


RB = 1000


def _mlp_kernel(h_ref, a0_ref, a1_ref, w1_ref, b1_ref, w2_ref, b2_ref, out_ref):
    z = h_ref[...] + a0_ref[...] + a1_ref[...]
    y = jnp.dot(z, w1_ref[...], preferred_element_type=jnp.float32) + b1_ref[...]
    y = jnp.maximum(y, 0.0)
    o = jnp.dot(y, w2_ref[...], preferred_element_type=jnp.float32) + b2_ref[...]
    out_ref[...] = jnp.maximum(o, 0.0)


def _mlp(h, a0, a1, w1, b1, w2, b2):
    grid = N // RB
    return pl.pallas_call(
        _mlp_kernel,
        grid=(grid,),
        in_specs=[
            pl.BlockSpec((RB, D), lambda i: (i, 0)),
            pl.BlockSpec((RB, D), lambda i: (i, 0)),
            pl.BlockSpec((RB, D), lambda i: (i, 0)),
            pl.BlockSpec((D, D), lambda i: (0, 0)),
            pl.BlockSpec((1, D), lambda i: (0, 0)),
            pl.BlockSpec((D, D), lambda i: (0, 0)),
            pl.BlockSpec((1, D), lambda i: (0, 0)),
        ],
        out_specs=pl.BlockSpec((RB, D), lambda i: (i, 0)),
        out_shape=jax.ShapeDtypeStruct((N, D), jnp.float32),
    )(h, a0, a1, w1, b1.reshape(1, D), w2, b2.reshape(1, D))


def _pool_kernel(batch_ref, h1_ref, h2_ref, h3_ref,
                 w1a_ref, w1b_ref, w1c_ref, bc1_ref,
                 gb_ref, wc2_ref, bc2_ref, out_ref,
                 acc1, acc2, acc3):
    i = pl.program_id(0)

    @pl.when(i == 0)
    def _init():
        acc1[...] = jnp.zeros_like(acc1)
        acc2[...] = jnp.zeros_like(acc2)
        acc3[...] = jnp.zeros_like(acc3)

    ids = batch_ref[0, 0, :]
    gids = lax.broadcasted_iota(jnp.int32, (G, RB), 0)
    onehot_t = (gids == ids[None, :]).astype(jnp.float32)
    acc1[...] += jnp.dot(onehot_t, h1_ref[...], preferred_element_type=jnp.float32)
    acc2[...] += jnp.dot(onehot_t, h2_ref[...], preferred_element_type=jnp.float32)
    acc3[...] += jnp.dot(onehot_t, h3_ref[...], preferred_element_type=jnp.float32)

    @pl.when(i == (N // RB) - 1)
    def _final():
        z = (jnp.dot(acc1[...], w1a_ref[...], preferred_element_type=jnp.float32)
             + jnp.dot(acc2[...], w1b_ref[...], preferred_element_type=jnp.float32)
             + jnp.dot(acc3[...], w1c_ref[...], preferred_element_type=jnp.float32)
             + bc1_ref[...])
        gamma = gb_ref[0:1, :]
        beta = gb_ref[1:2, :]
        z = gamma * z * (1.0 / jnp.sqrt(1.0 + 1e-5)) + beta
        z = jnp.maximum(z, 0.0)
        out_ref[...] = jnp.dot(z, wc2_ref[...], preferred_element_type=jnp.float32) + bc2_ref[...]


def _pool_classify(batch3d, h1, h2, h3, w1a, w1b, w1c, bc1, gb, wc2p, bc2p):
    grid = N // RB
    return pl.pallas_call(
        _pool_kernel,
        grid=(grid,),
        in_specs=[
            pl.BlockSpec((1, 1, RB), lambda i: (i, 0, 0)),
            pl.BlockSpec((RB, D), lambda i: (i, 0)),
            pl.BlockSpec((RB, D), lambda i: (i, 0)),
            pl.BlockSpec((RB, D), lambda i: (i, 0)),
            pl.BlockSpec((D, D), lambda i: (0, 0)),
            pl.BlockSpec((D, D), lambda i: (0, 0)),
            pl.BlockSpec((D, D), lambda i: (0, 0)),
            pl.BlockSpec((1, D), lambda i: (0, 0)),
            pl.BlockSpec((2, D), lambda i: (0, 0)),
            pl.BlockSpec((D, D), lambda i: (0, 0)),
            pl.BlockSpec((1, D), lambda i: (0, 0)),
        ],
        out_specs=pl.BlockSpec((G, D), lambda i: (0, 0)),
        out_shape=jax.ShapeDtypeStruct((G, D), jnp.float32),
        scratch_shapes=[
            pltpu.VMEM((G, D), jnp.float32),
            pltpu.VMEM((G, D), jnp.float32),
            pltpu.VMEM((G, D), jnp.float32),
        ],
    )(batch3d, h1, h2, h3, w1a, w1b, w1c, bc1, gb, wc2p, bc2p)


def kernel(x, edge_index, edge_attr, batch,
           W1_0, b1_0, W2_0, b2_0,
           W1_1, b1_1, W2_1, b2_1,
           W1_2, b1_2, W2_2, b2_2,
           Wc1, bc1, gamma, beta, Wc2, bc2):
    del edge_attr

    src3d = edge_index[0].reshape(NW, NCHUNK, CH)
    dst3d = edge_index[1].reshape(NW, NCHUNK, CH)
    idx4d = jnp.stack([src3d, dst3d], axis=2)
    zeros = jnp.zeros((RPT, D), dtype=jnp.float32)

    params = [(W1_0, b1_0, W2_0, b2_0),
              (W1_1, b1_1, W2_1, b2_1),
              (W1_2, b1_2, W2_2, b2_2)]

    h = x
    hs = []
    for (w1, b1, w2, b2) in params:
        agg = _sc_agg(h, idx4d, zeros)
        h = _mlp(h, agg[:N], agg[NPAD:NPAD + N], w1, b1, w2, b2)
        hs.append(h)

    batch3d = batch.reshape(N // RB, 1, RB)
    w1a = Wc1[0:D]
    w1b = Wc1[D:2 * D]
    w1c = Wc1[2 * D:]
    gb = jnp.stack([gamma, beta], axis=0)
    wc2p = jnp.zeros((D, D), jnp.float32).at[:, :Wc2.shape[1]].set(Wc2)
    bc2p = jnp.zeros((1, D), jnp.float32).at[0, :Wc2.shape[1]].set(bc2)

    out = _pool_classify(batch3d, hs[0], hs[1], hs[2],
                         w1a, w1b, w1c, bc1.reshape(1, D), gb, wc2p, bc2p)
    return out[:, :Wc2.shape[1]]


# speedup vs baseline: 9.9773x; 1.0637x over previous
import functools

import jax
import jax.numpy as jnp
from jax import lax
from jax.experimental import pallas as pl
from jax.experimental.pallas import tpu as pltpu
from jax.experimental.pallas import tpu_sc as plsc

N = 10000
E = 320000
D = 128
G = 128

NC = 2
NS = 16
NW = NC * NS
EPW = E // NW
CH = 100
NCHUNK = EPW // CH
DEPTH = 3
NPAD = 10240
RPT = NPAD // NS


def _sc_agg_kernel(x_hbm, idx_hbm, zeros_hbm, out_hbm,
                   idx_v, rows_v, acc_sh,
                   gsem0, gsem1, gsem2, isem0, isem1, isem2):
    c = lax.axis_index("c")
    s = lax.axis_index("s")
    w = s * NC + c

    pltpu.sync_copy(zeros_hbm, acc_sh.at[pl.ds(s * RPT, RPT)])

    plsc.subcore_barrier()

    rows = tuple(rows_v.at[pl.ds(k * CH, CH)] for k in range(DEPTH))
    islot = tuple(idx_v.at[k] for k in range(DEPTH))
    gsem = (gsem0, gsem1, gsem2)
    isem = (isem0, isem1, isem2)

    def stage_idx(j, p):
        pltpu.async_copy(idx_hbm.at[w].at[j], islot[p].at[0], isem[p])
        pltpu.async_copy(idx_hbm.at[NW + w].at[j], islot[p].at[1], isem[p])

    def wait_idx(j, p):
        pltpu.make_async_copy(idx_hbm.at[w].at[j], islot[p].at[0], isem[p]).wait()
        pltpu.make_async_copy(idx_hbm.at[NW + w].at[j], islot[p].at[1], isem[p]).wait()

    def gather(p):
        pltpu.async_copy(x_hbm.at[islot[p].at[0]], rows[p], gsem[p])

    def wait_gather(p):
        pltpu.make_async_copy(x_hbm.at[islot[p].at[0]], rows[p], gsem[p]).wait()

    def scatter(p):
        pltpu.sync_copy(rows[p], acc_sh.at[islot[p].at[1]], add=True)

    stage_idx(0, 0)
    stage_idx(1, 1)
    wait_idx(0, 0)
    gather(0)
    wait_idx(1, 1)
    gather(1)
    stage_idx(2, 2)

    def step(j, a, do_gather=True, do_stage=True):
        if do_gather:
            cslot = (a + 2) % DEPTH
            wait_idx(j + 2, cslot)
            gather(cslot)
        wait_gather(a)
        scatter(a)
        if do_stage:
            stage_idx(j + 3, a)

    def body(j3, carry):
        j = j3 * DEPTH
        step(j, 0)
        step(j + 1, 1)
        step(j + 2, 2)
        return carry

    lax.fori_loop(0, (NCHUNK - 4) // DEPTH, body, 0)
    step(NCHUNK - 4, (NCHUNK - 4) % DEPTH)
    step(NCHUNK - 3, (NCHUNK - 3) % DEPTH, do_stage=False)
    step(NCHUNK - 2, (NCHUNK - 2) % DEPTH, do_gather=False, do_stage=False)
    step(NCHUNK - 1, (NCHUNK - 1) % DEPTH, do_gather=False, do_stage=False)

    plsc.subcore_barrier()

    pltpu.sync_copy(acc_sh.at[pl.ds(s * RPT, RPT)],
                    out_hbm.at[pl.ds(c * NPAD + s * RPT, RPT)])


@functools.partial(jax.jit, static_argnames=())
def _sc_agg(x, idx4d, zeros):
    mesh = plsc.VectorSubcoreMesh(core_axis_name="c", subcore_axis_name="s",
                                  num_cores=NC, num_subcores=NS)
    f = pl.kernel(
        _sc_agg_kernel,
        out_type=jax.ShapeDtypeStruct((NC * NPAD, D), jnp.float32),
        mesh=mesh,
        scratch_types=[
            pltpu.VMEM((DEPTH, 2, CH), jnp.int32),
            pltpu.VMEM((DEPTH * CH, D), jnp.float32),
            pltpu.VMEM_SHARED((NPAD, D), jnp.float32),
            pltpu.SemaphoreType.DMA,
            pltpu.SemaphoreType.DMA,
            pltpu.SemaphoreType.DMA,
            pltpu.SemaphoreType.DMA,
            pltpu.SemaphoreType.DMA,
            pltpu.SemaphoreType.DMA,
        ],
    )
    return f(x, idx4d, zeros)


RB = 1000


def _mlp_kernel(batch_ref, h_ref, a0_ref, a1_ref, w1_ref, b1_ref,
                w2_ref, b2_ref, out_ref, pool_ref, acc):
    i = pl.program_id(0)
    z = h_ref[...] + a0_ref[...] + a1_ref[...]
    y = jnp.dot(z, w1_ref[...], preferred_element_type=jnp.float32) + b1_ref[...]
    y = jnp.maximum(y, 0.0)
    o = jnp.dot(y, w2_ref[...], preferred_element_type=jnp.float32) + b2_ref[...]
    o = jnp.maximum(o, 0.0)
    out_ref[...] = o

    ids = batch_ref[0, 0, :]
    gids = lax.broadcasted_iota(jnp.int32, (G, RB), 0)
    onehot_t = (gids == ids[None, :]).astype(jnp.float32)
    contrib = jnp.dot(onehot_t, o, preferred_element_type=jnp.float32)

    @pl.when(i == 0)
    def _init():
        acc[...] = jnp.zeros_like(acc)

    acc[...] += contrib

    @pl.when(i == (N // RB) - 1)
    def _flush():
        pool_ref[...] = acc[...]


def _mlp(batch3d, h, a0, a1, w1, b1, w2, b2):
    grid = N // RB
    return pl.pallas_call(
        _mlp_kernel,
        grid=(grid,),
        in_specs=[
            pl.BlockSpec((1, 1, RB), lambda i: (i, 0, 0)),
            pl.BlockSpec((RB, D), lambda i: (i, 0)),
            pl.BlockSpec((RB, D), lambda i: (i, 0)),
            pl.BlockSpec((RB, D), lambda i: (i, 0)),
            pl.BlockSpec((D, D), lambda i: (0, 0)),
            pl.BlockSpec((1, D), lambda i: (0, 0)),
            pl.BlockSpec((D, D), lambda i: (0, 0)),
            pl.BlockSpec((1, D), lambda i: (0, 0)),
        ],
        out_specs=[
            pl.BlockSpec((RB, D), lambda i: (i, 0)),
            pl.BlockSpec((G, D), lambda i: (0, 0)),
        ],
        out_shape=[
            jax.ShapeDtypeStruct((N, D), jnp.float32),
            jax.ShapeDtypeStruct((G, D), jnp.float32),
        ],
        scratch_shapes=[pltpu.VMEM((G, D), jnp.float32)],
    )(batch3d, h, a0, a1, w1, b1.reshape(1, D), w2, b2.reshape(1, D))


def _cls_kernel(p1_ref, p2_ref, p3_ref, w1a_ref, w1b_ref, w1c_ref, bc1_ref,
                gb_ref, wc2_ref, bc2_ref, out_ref):
    z = (jnp.dot(p1_ref[...], w1a_ref[...], preferred_element_type=jnp.float32)
         + jnp.dot(p2_ref[...], w1b_ref[...], preferred_element_type=jnp.float32)
         + jnp.dot(p3_ref[...], w1c_ref[...], preferred_element_type=jnp.float32)
         + bc1_ref[...])
    gamma = gb_ref[0:1, :]
    beta = gb_ref[1:2, :]
    z = gamma * z * (1.0 / jnp.sqrt(1.0 + 1e-5)) + beta
    z = jnp.maximum(z, 0.0)
    out_ref[...] = jnp.dot(z, wc2_ref[...], preferred_element_type=jnp.float32) + bc2_ref[...]


def _classify(p1, p2, p3, w1a, w1b, w1c, bc1, gb, wc2p, bc2p):
    return pl.pallas_call(
        _cls_kernel,
        out_shape=jax.ShapeDtypeStruct((G, D), jnp.float32),
    )(p1, p2, p3, w1a, w1b, w1c, bc1, gb, wc2p, bc2p)


def kernel(x, edge_index, edge_attr, batch,
           W1_0, b1_0, W2_0, b2_0,
           W1_1, b1_1, W2_1, b2_1,
           W1_2, b1_2, W2_2, b2_2,
           Wc1, bc1, gamma, beta, Wc2, bc2):
    del edge_attr

    eidx3d = edge_index.reshape(2 * NW, NCHUNK, CH)
    zeros = jnp.zeros((RPT, D), dtype=jnp.float32)
    batch3d = batch.reshape(N // RB, 1, RB)

    params = [(W1_0, b1_0, W2_0, b2_0),
              (W1_1, b1_1, W2_1, b2_1),
              (W1_2, b1_2, W2_2, b2_2)]

    h = x
    pooled = []
    for (w1, b1, w2, b2) in params:
        agg = _sc_agg(h, eidx3d, zeros)
        h, p = _mlp(batch3d, h, agg[:N], agg[NPAD:NPAD + N], w1, b1, w2, b2)
        pooled.append(p)

    w1a = Wc1[0:D]
    w1b = Wc1[D:2 * D]
    w1c = Wc1[2 * D:]
    gb = jnp.stack([gamma, beta], axis=0)
    wc2p = jnp.zeros((D, D), jnp.float32).at[:, :Wc2.shape[1]].set(Wc2)
    bc2p = jnp.zeros((1, D), jnp.float32).at[0, :Wc2.shape[1]].set(bc2)

    out = _classify(pooled[0], pooled[1], pooled[2],
                    w1a, w1b, w1c, bc1.reshape(1, D), gb, wc2p, bc2p)
    return out[:, :Wc2.shape[1]]


# speedup vs baseline: 10.0731x; 1.0096x over previous
import functools

import jax
import jax.numpy as jnp
from jax import lax
from jax.experimental import pallas as pl
from jax.experimental.pallas import tpu as pltpu
from jax.experimental.pallas import tpu_sc as plsc

N = 10000
E = 320000
D = 128
G = 128

NC = 2
NS = 16
NW = NC * NS
EPW = E // NW
CH = 100
NCHUNK = EPW // CH
DEPTH = 3
NPAD = 10240
RPT = NPAD // NS


def _sc_agg_kernel(x_hbm, idx_hbm, zeros_hbm, out_hbm,
                   idx_v, rows_v, acc_sh,
                   gsem0, gsem1, gsem2, isem0, isem1, isem2, zsem):
    c = lax.axis_index("c")
    s = lax.axis_index("s")
    w = s * NC + c

    pltpu.async_copy(zeros_hbm, acc_sh.at[pl.ds(s * RPT, RPT)], zsem)

    rows = tuple(rows_v.at[pl.ds(k * CH, CH)] for k in range(DEPTH))
    islot = tuple(idx_v.at[k] for k in range(DEPTH))
    gsem = (gsem0, gsem1, gsem2)
    isem = (isem0, isem1, isem2)

    def stage_idx(j, p):
        pltpu.async_copy(idx_hbm.at[w].at[j], islot[p].at[0], isem[p])
        pltpu.async_copy(idx_hbm.at[NW + w].at[j], islot[p].at[1], isem[p])

    def wait_idx(j, p):
        pltpu.make_async_copy(idx_hbm.at[w].at[j], islot[p].at[0], isem[p]).wait()
        pltpu.make_async_copy(idx_hbm.at[NW + w].at[j], islot[p].at[1], isem[p]).wait()

    def gather(p):
        pltpu.async_copy(x_hbm.at[islot[p].at[0]], rows[p], gsem[p])

    def wait_gather(p):
        pltpu.make_async_copy(x_hbm.at[islot[p].at[0]], rows[p], gsem[p]).wait()

    def scatter(p):
        pltpu.sync_copy(rows[p], acc_sh.at[islot[p].at[1]], add=True)

    stage_idx(0, 0)
    stage_idx(1, 1)
    wait_idx(0, 0)
    gather(0)
    wait_idx(1, 1)
    gather(1)
    stage_idx(2, 2)
    pltpu.make_async_copy(zeros_hbm, acc_sh.at[pl.ds(s * RPT, RPT)], zsem).wait()
    plsc.subcore_barrier()

    def step(j, a, do_gather=True, do_stage=True):
        if do_gather:
            cslot = (a + 2) % DEPTH
            wait_idx(j + 2, cslot)
            gather(cslot)
        wait_gather(a)
        scatter(a)
        if do_stage:
            stage_idx(j + 3, a)

    def body(j3, carry):
        j = j3 * DEPTH
        step(j, 0)
        step(j + 1, 1)
        step(j + 2, 2)
        return carry

    lax.fori_loop(0, (NCHUNK - 4) // DEPTH, body, 0)
    step(NCHUNK - 4, (NCHUNK - 4) % DEPTH)
    step(NCHUNK - 3, (NCHUNK - 3) % DEPTH, do_stage=False)
    step(NCHUNK - 2, (NCHUNK - 2) % DEPTH, do_gather=False, do_stage=False)
    step(NCHUNK - 1, (NCHUNK - 1) % DEPTH, do_gather=False, do_stage=False)

    plsc.subcore_barrier()

    pltpu.sync_copy(acc_sh.at[pl.ds(s * RPT, RPT)],
                    out_hbm.at[pl.ds(c * NPAD + s * RPT, RPT)])


@functools.partial(jax.jit, static_argnames=())
def _sc_agg(x, idx4d, zeros):
    mesh = plsc.VectorSubcoreMesh(core_axis_name="c", subcore_axis_name="s",
                                  num_cores=NC, num_subcores=NS)
    f = pl.kernel(
        _sc_agg_kernel,
        out_type=jax.ShapeDtypeStruct((NC * NPAD, D), jnp.float32),
        mesh=mesh,
        scratch_types=[
            pltpu.VMEM((DEPTH, 2, CH), jnp.int32),
            pltpu.VMEM((DEPTH * CH, D), jnp.float32),
            pltpu.VMEM_SHARED((NPAD, D), jnp.float32),
            pltpu.SemaphoreType.DMA,
            pltpu.SemaphoreType.DMA,
            pltpu.SemaphoreType.DMA,
            pltpu.SemaphoreType.DMA,
            pltpu.SemaphoreType.DMA,
            pltpu.SemaphoreType.DMA,
            pltpu.SemaphoreType.DMA,
        ],
    )
    return f(x, idx4d, zeros)


RB = 1000


def _mlp_kernel(batch_ref, h_ref, a0_ref, a1_ref, w1_ref, b1_ref,
                w2_ref, b2_ref, out_ref, pool_ref, acc):
    i = pl.program_id(0)
    z = h_ref[...] + a0_ref[...] + a1_ref[...]
    y = jnp.dot(z, w1_ref[...], preferred_element_type=jnp.float32) + b1_ref[...]
    y = jnp.maximum(y, 0.0)
    o = jnp.dot(y, w2_ref[...], preferred_element_type=jnp.float32) + b2_ref[...]
    o = jnp.maximum(o, 0.0)
    out_ref[...] = o

    ids = batch_ref[0, 0, :]
    gids = lax.broadcasted_iota(jnp.int32, (G, RB), 0)
    onehot_t = (gids == ids[None, :]).astype(jnp.float32)
    contrib = jnp.dot(onehot_t, o, preferred_element_type=jnp.float32)

    @pl.when(i == 0)
    def _init():
        acc[...] = jnp.zeros_like(acc)

    acc[...] += contrib

    @pl.when(i == (N // RB) - 1)
    def _flush():
        pool_ref[...] = acc[...]


def _mlp(batch3d, h, a0, a1, w1, b1, w2, b2):
    grid = N // RB
    return pl.pallas_call(
        _mlp_kernel,
        grid=(grid,),
        in_specs=[
            pl.BlockSpec((1, 1, RB), lambda i: (i, 0, 0)),
            pl.BlockSpec((RB, D), lambda i: (i, 0)),
            pl.BlockSpec((RB, D), lambda i: (i, 0)),
            pl.BlockSpec((RB, D), lambda i: (i, 0)),
            pl.BlockSpec((D, D), lambda i: (0, 0)),
            pl.BlockSpec((1, D), lambda i: (0, 0)),
            pl.BlockSpec((D, D), lambda i: (0, 0)),
            pl.BlockSpec((1, D), lambda i: (0, 0)),
        ],
        out_specs=[
            pl.BlockSpec((RB, D), lambda i: (i, 0)),
            pl.BlockSpec((G, D), lambda i: (0, 0)),
        ],
        out_shape=[
            jax.ShapeDtypeStruct((N, D), jnp.float32),
            jax.ShapeDtypeStruct((G, D), jnp.float32),
        ],
        scratch_shapes=[pltpu.VMEM((G, D), jnp.float32)],
    )(batch3d, h, a0, a1, w1, b1.reshape(1, D), w2, b2.reshape(1, D))


def _cls_kernel(p1_ref, p2_ref, p3_ref, w1a_ref, w1b_ref, w1c_ref, bc1_ref,
                gb_ref, wc2_ref, bc2_ref, out_ref):
    z = (jnp.dot(p1_ref[...], w1a_ref[...], preferred_element_type=jnp.float32)
         + jnp.dot(p2_ref[...], w1b_ref[...], preferred_element_type=jnp.float32)
         + jnp.dot(p3_ref[...], w1c_ref[...], preferred_element_type=jnp.float32)
         + bc1_ref[...])
    gamma = gb_ref[0:1, :]
    beta = gb_ref[1:2, :]
    z = gamma * z * (1.0 / jnp.sqrt(1.0 + 1e-5)) + beta
    z = jnp.maximum(z, 0.0)
    out_ref[...] = jnp.dot(z, wc2_ref[...], preferred_element_type=jnp.float32) + bc2_ref[...]


def _classify(p1, p2, p3, w1a, w1b, w1c, bc1, gb, wc2p, bc2p):
    return pl.pallas_call(
        _cls_kernel,
        out_shape=jax.ShapeDtypeStruct((G, D), jnp.float32),
    )(p1, p2, p3, w1a, w1b, w1c, bc1, gb, wc2p, bc2p)


def kernel(x, edge_index, edge_attr, batch,
           W1_0, b1_0, W2_0, b2_0,
           W1_1, b1_1, W2_1, b2_1,
           W1_2, b1_2, W2_2, b2_2,
           Wc1, bc1, gamma, beta, Wc2, bc2):
    del edge_attr

    eidx3d = edge_index.reshape(2 * NW, NCHUNK, CH)
    zeros = jnp.zeros((RPT, D), dtype=jnp.float32)
    batch3d = batch.reshape(N // RB, 1, RB)

    params = [(W1_0, b1_0, W2_0, b2_0),
              (W1_1, b1_1, W2_1, b2_1),
              (W1_2, b1_2, W2_2, b2_2)]

    h = x
    pooled = []
    for (w1, b1, w2, b2) in params:
        agg = _sc_agg(h, eidx3d, zeros)
        h, p = _mlp(batch3d, h, agg[:N], agg[NPAD:NPAD + N], w1, b1, w2, b2)
        pooled.append(p)

    w1a = Wc1[0:D]
    w1b = Wc1[D:2 * D]
    w1c = Wc1[2 * D:]
    gb = jnp.stack([gamma, beta], axis=0)
    wc2p = jnp.zeros((D, D), jnp.float32).at[:, :Wc2.shape[1]].set(Wc2)
    bc2p = jnp.zeros((1, D), jnp.float32).at[0, :Wc2.shape[1]].set(bc2)

    out = _classify(pooled[0], pooled[1], pooled[2],
                    w1a, w1b, w1c, bc1.reshape(1, D), gb, wc2p, bc2p)
    return out[:, :Wc2.shape[1]]


# speedup vs baseline: 10.1048x; 1.0032x over previous
import functools

import jax
import jax.numpy as jnp
from jax import lax
from jax.experimental import pallas as pl
from jax.experimental.pallas import tpu as pltpu
from jax.experimental.pallas import tpu_sc as plsc

N = 10000
E = 320000
D = 128
G = 128

NC = 2
NS = 16
NW = NC * NS
EPW = E // NW
CH = 100
NCHUNK = EPW // CH
DEPTH = 3
NPAD = 10240
RPT = NPAD // NS


def _sc_agg_kernel(x_hbm, idx_hbm, zeros_hbm, out_hbm,
                   idx_v, rows_v, acc_sh,
                   gsem0, gsem1, gsem2, isem0, isem1, isem2, zsem):
    c = lax.axis_index("c")
    s = lax.axis_index("s")
    w = s * NC + c

    pltpu.async_copy(zeros_hbm, acc_sh.at[pl.ds(s * RPT, RPT)], zsem)

    rows = tuple(rows_v.at[pl.ds(k * CH, CH)] for k in range(DEPTH))
    islot = tuple(idx_v.at[k] for k in range(DEPTH))
    gsem = (gsem0, gsem1, gsem2)
    isem = (isem0, isem1, isem2)

    def stage_idx(j, p):
        pltpu.async_copy(idx_hbm.at[w].at[j], islot[p].at[0], isem[p])
        pltpu.async_copy(idx_hbm.at[NW + w].at[j], islot[p].at[1], isem[p])

    def wait_idx(j, p):
        pltpu.make_async_copy(idx_hbm.at[w].at[j], islot[p].at[0], isem[p]).wait()
        pltpu.make_async_copy(idx_hbm.at[NW + w].at[j], islot[p].at[1], isem[p]).wait()

    def gather(p):
        pltpu.async_copy(x_hbm.at[islot[p].at[0]], rows[p], gsem[p])

    def wait_gather(p):
        pltpu.make_async_copy(x_hbm.at[islot[p].at[0]], rows[p], gsem[p]).wait()

    def scatter(p):
        pltpu.sync_copy(rows[p], acc_sh.at[islot[p].at[1]], add=True)

    stage_idx(0, 0)
    stage_idx(1, 1)
    wait_idx(0, 0)
    gather(0)
    wait_idx(1, 1)
    gather(1)
    stage_idx(2, 2)
    pltpu.make_async_copy(zeros_hbm, acc_sh.at[pl.ds(s * RPT, RPT)], zsem).wait()
    plsc.subcore_barrier()

    def step(j, a, do_gather=True, do_stage=True):
        if do_gather:
            cslot = (a + 2) % DEPTH
            wait_idx(j + 2, cslot)
            gather(cslot)
        wait_gather(a)
        scatter(a)
        if do_stage:
            stage_idx(j + 3, a)

    def body(j3, carry):
        j = j3 * DEPTH
        step(j, 0)
        step(j + 1, 1)
        step(j + 2, 2)
        return carry

    lax.fori_loop(0, (NCHUNK - 4) // DEPTH, body, 0)
    step(NCHUNK - 4, (NCHUNK - 4) % DEPTH)
    step(NCHUNK - 3, (NCHUNK - 3) % DEPTH, do_stage=False)
    step(NCHUNK - 2, (NCHUNK - 2) % DEPTH, do_gather=False, do_stage=False)
    step(NCHUNK - 1, (NCHUNK - 1) % DEPTH, do_gather=False, do_stage=False)

    plsc.subcore_barrier()

    pltpu.sync_copy(acc_sh.at[pl.ds(s * RPT, RPT)],
                    out_hbm.at[pl.ds(c * NPAD + s * RPT, RPT)])


@functools.partial(jax.jit, static_argnames=())
def _sc_agg(x, idx4d, zeros):
    mesh = plsc.VectorSubcoreMesh(core_axis_name="c", subcore_axis_name="s",
                                  num_cores=NC, num_subcores=NS)
    f = pl.kernel(
        _sc_agg_kernel,
        out_type=jax.ShapeDtypeStruct((NC * NPAD, D), jnp.float32),
        mesh=mesh,
        scratch_types=[
            pltpu.VMEM((DEPTH, 2, CH), jnp.int32),
            pltpu.VMEM((DEPTH * CH, D), jnp.float32),
            pltpu.VMEM_SHARED((NPAD, D), jnp.float32),
            pltpu.SemaphoreType.DMA,
            pltpu.SemaphoreType.DMA,
            pltpu.SemaphoreType.DMA,
            pltpu.SemaphoreType.DMA,
            pltpu.SemaphoreType.DMA,
            pltpu.SemaphoreType.DMA,
            pltpu.SemaphoreType.DMA,
        ],
    )
    return f(x, idx4d, zeros)


RB = 1000


def _mlp_kernel(batch_ref, h_ref, a0_ref, a1_ref, w1_ref, b1_ref,
                w2_ref, b2_ref, out_ref, pool_ref, acc):
    i = pl.program_id(0)
    z = h_ref[...] + a0_ref[...] + a1_ref[...]
    y = jnp.dot(z, w1_ref[...], preferred_element_type=jnp.float32) + b1_ref[...]
    y = jnp.maximum(y, 0.0)
    o = jnp.dot(y, w2_ref[...], preferred_element_type=jnp.float32) + b2_ref[...]
    o = jnp.maximum(o, 0.0)
    out_ref[...] = o

    ids = batch_ref[0, 0, :]
    gids = lax.broadcasted_iota(jnp.int32, (G, RB), 0)
    onehot_t = (gids == ids[None, :]).astype(jnp.float32)
    contrib = jnp.dot(onehot_t, o, preferred_element_type=jnp.float32)

    @pl.when(i == 0)
    def _init():
        acc[...] = jnp.zeros_like(acc)

    acc[...] += contrib

    @pl.when(i == (N // RB) - 1)
    def _flush():
        pool_ref[...] = acc[...]


def _mlp(batch3d, h, a0, a1, w1, b1, w2, b2):
    grid = N // RB
    return pl.pallas_call(
        _mlp_kernel,
        grid=(grid,),
        in_specs=[
            pl.BlockSpec((1, 1, RB), lambda i: (i, 0, 0)),
            pl.BlockSpec((RB, D), lambda i: (i, 0)),
            pl.BlockSpec((RB, D), lambda i: (i, 0)),
            pl.BlockSpec((RB, D), lambda i: (i, 0)),
            pl.BlockSpec((D, D), lambda i: (0, 0)),
            pl.BlockSpec((1, D), lambda i: (0, 0)),
            pl.BlockSpec((D, D), lambda i: (0, 0)),
            pl.BlockSpec((1, D), lambda i: (0, 0)),
        ],
        out_specs=[
            pl.BlockSpec((RB, D), lambda i: (i, 0)),
            pl.BlockSpec((G, D), lambda i: (0, 0)),
        ],
        out_shape=[
            jax.ShapeDtypeStruct((N, D), jnp.float32),
            jax.ShapeDtypeStruct((G, D), jnp.float32),
        ],
        scratch_shapes=[pltpu.VMEM((G, D), jnp.float32)],
    )(batch3d, h, a0, a1, w1, b1.reshape(1, D), w2, b2.reshape(1, D))


def _mlp_final_kernel(batch_ref, h_ref, a0_ref, a1_ref, w1_ref, b1_ref,
                      w2_ref, b2_ref, p1_ref, p2_ref,
                      w1a_ref, w1b_ref, w1c_ref, bc1_ref,
                      gb_ref, wc2_ref, bc2_ref, out_ref, acc):
    i = pl.program_id(0)
    z = h_ref[...] + a0_ref[...] + a1_ref[...]
    y = jnp.dot(z, w1_ref[...], preferred_element_type=jnp.float32) + b1_ref[...]
    y = jnp.maximum(y, 0.0)
    o = jnp.dot(y, w2_ref[...], preferred_element_type=jnp.float32) + b2_ref[...]
    o = jnp.maximum(o, 0.0)

    ids = batch_ref[0, 0, :]
    gids = lax.broadcasted_iota(jnp.int32, (G, RB), 0)
    onehot_t = (gids == ids[None, :]).astype(jnp.float32)
    contrib = jnp.dot(onehot_t, o, preferred_element_type=jnp.float32)

    @pl.when(i == 0)
    def _init():
        acc[...] = jnp.zeros_like(acc)

    acc[...] += contrib

    @pl.when(i == (N // RB) - 1)
    def _final():
        zc = (jnp.dot(p1_ref[...], w1a_ref[...], preferred_element_type=jnp.float32)
              + jnp.dot(p2_ref[...], w1b_ref[...], preferred_element_type=jnp.float32)
              + jnp.dot(acc[...], w1c_ref[...], preferred_element_type=jnp.float32)
              + bc1_ref[...])
        gamma = gb_ref[0:1, :]
        beta = gb_ref[1:2, :]
        zc = gamma * zc * (1.0 / jnp.sqrt(1.0 + 1e-5)) + beta
        zc = jnp.maximum(zc, 0.0)
        out_ref[...] = jnp.dot(zc, wc2_ref[...], preferred_element_type=jnp.float32) + bc2_ref[...]


def _mlp_final(batch3d, h, a0, a1, w1, b1, w2, b2, p1, p2,
               w1a, w1b, w1c, bc1, gb, wc2p, bc2p):
    grid = N // RB
    full = lambda i: (0, 0)
    return pl.pallas_call(
        _mlp_final_kernel,
        grid=(grid,),
        in_specs=[
            pl.BlockSpec((1, 1, RB), lambda i: (i, 0, 0)),
            pl.BlockSpec((RB, D), lambda i: (i, 0)),
            pl.BlockSpec((RB, D), lambda i: (i, 0)),
            pl.BlockSpec((RB, D), lambda i: (i, 0)),
            pl.BlockSpec((D, D), full),
            pl.BlockSpec((1, D), full),
            pl.BlockSpec((D, D), full),
            pl.BlockSpec((1, D), full),
            pl.BlockSpec((G, D), full),
            pl.BlockSpec((G, D), full),
            pl.BlockSpec((D, D), full),
            pl.BlockSpec((D, D), full),
            pl.BlockSpec((D, D), full),
            pl.BlockSpec((1, D), full),
            pl.BlockSpec((2, D), full),
            pl.BlockSpec((D, D), full),
            pl.BlockSpec((1, D), full),
        ],
        out_specs=pl.BlockSpec((G, D), full),
        out_shape=jax.ShapeDtypeStruct((G, D), jnp.float32),
        scratch_shapes=[pltpu.VMEM((G, D), jnp.float32)],
    )(batch3d, h, a0, a1, w1, b1.reshape(1, D), w2, b2.reshape(1, D),
      p1, p2, w1a, w1b, w1c, bc1, gb, wc2p, bc2p)


def kernel(x, edge_index, edge_attr, batch,
           W1_0, b1_0, W2_0, b2_0,
           W1_1, b1_1, W2_1, b2_1,
           W1_2, b1_2, W2_2, b2_2,
           Wc1, bc1, gamma, beta, Wc2, bc2):
    del edge_attr

    eidx3d = edge_index.reshape(2 * NW, NCHUNK, CH)
    zeros = jnp.zeros((RPT, D), dtype=jnp.float32)
    batch3d = batch.reshape(N // RB, 1, RB)

    params = [(W1_0, b1_0, W2_0, b2_0),
              (W1_1, b1_1, W2_1, b2_1),
              (W1_2, b1_2, W2_2, b2_2)]

    w1a = Wc1[0:D]
    w1b = Wc1[D:2 * D]
    w1c = Wc1[2 * D:]
    gb = jnp.stack([gamma, beta], axis=0)
    wc2p = jnp.zeros((D, D), jnp.float32).at[:, :Wc2.shape[1]].set(Wc2)
    bc2p = jnp.zeros((1, D), jnp.float32).at[0, :Wc2.shape[1]].set(bc2)

    h = x
    pooled = []
    for (w1, b1, w2, b2) in params[:2]:
        agg = _sc_agg(h, eidx3d, zeros)
        h, p = _mlp(batch3d, h, agg[:N], agg[NPAD:NPAD + N], w1, b1, w2, b2)
        pooled.append(p)

    (w1, b1, w2, b2) = params[2]
    agg = _sc_agg(h, eidx3d, zeros)
    out = _mlp_final(batch3d, h, agg[:N], agg[NPAD:NPAD + N], w1, b1, w2, b2,
                     pooled[0], pooled[1], w1a, w1b, w1c,
                     bc1.reshape(1, D), gb, wc2p, bc2p)
    return out[:, :Wc2.shape[1]]
